# Initial kernel scaffold; baseline (speedup 1.0000x reference)
import jax
import jax.numpy as jnp
from jax.experimental import pallas as pl


def kernel(relative_position_bias_table, relative_position_index):
    raise NotImplementedError("write your pallas kernel here")


# speedup vs baseline: 12.3486x; 12.3486x over previous
import functools

import jax
import jax.numpy as jnp
from jax import lax
from jax.experimental import pallas as pl
from jax.experimental.pallas import tpu as pltpu
from jax.experimental.pallas import tpu_sc as plsc

NC = 2
NS = 16
L = 16


@functools.lru_cache(maxsize=None)
def _build_sc_gather(num_heads: int, area2: int, v_pad: int):
  nw = NC * NS
  chunk = area2 // nw
  nvec = chunk // L

  mesh = plsc.VectorSubcoreMesh(core_axis_name="c", subcore_axis_name="s")

  @functools.partial(
      pl.kernel,
      out_type=jax.ShapeDtypeStruct((num_heads, area2), jnp.float32),
      mesh=mesh,
      scratch_types=[
          pltpu.VMEM((chunk,), jnp.int32),
          pltpu.VMEM((v_pad,), jnp.float32),
          pltpu.VMEM((chunk,), jnp.float32),
      ],
      compiler_params=pltpu.CompilerParams(needs_layout_passes=False),
  )
  def sc_gather(table_t_hbm, idx_hbm, out_hbm, idx_v, trow_v, out_v):
    wid = lax.axis_index("s") * NC + lax.axis_index("c")
    base = wid * chunk
    pltpu.sync_copy(idx_hbm.at[pl.ds(base, chunk)], idx_v)
    for h in range(num_heads):
      pltpu.sync_copy(table_t_hbm.at[h], trow_v)

      def body(v, _):
        iv = idx_v[pl.ds(v * L, L)]
        out_v[pl.ds(v * L, L)] = plsc.load_gather(trow_v, [iv])
        return None

      lax.fori_loop(0, nvec, body, None)
      pltpu.sync_copy(out_v, out_hbm.at[h, pl.ds(base, chunk)])

  return sc_gather


def kernel(relative_position_bias_table, relative_position_index):
  v, h = relative_position_bias_table.shape
  area = relative_position_index.shape[0]
  area2 = relative_position_index.size
  v_pad = -(-v // 8) * 8

  table_t = jnp.zeros((h, v_pad), jnp.float32)
  table_t = table_t.at[:, :v].set(relative_position_bias_table.T)
  idx_flat = relative_position_index.reshape(-1).astype(jnp.int32)

  out = _build_sc_gather(h, area2, v_pad)(table_t, idx_flat)
  return out.reshape(1, h, area, area)


# speedup vs baseline: 25.0383x; 2.0276x over previous
import functools

import jax
import jax.numpy as jnp
from jax import lax
from jax.experimental import pallas as pl
from jax.experimental.pallas import tpu as pltpu
from jax.experimental.pallas import tpu_sc as plsc

NC = 2
NS = 16
L = 16


@functools.lru_cache(maxsize=None)
def _build_sc_gather(num_heads: int, area2: int, v_pad: int):
  nw = NC * NS
  chunk = area2 // nw
  nvec = chunk // L

  mesh = plsc.VectorSubcoreMesh(core_axis_name="c", subcore_axis_name="s")

  @functools.partial(
      pl.kernel,
      out_type=jax.ShapeDtypeStruct((num_heads, area2), jnp.float32),
      mesh=mesh,
      scratch_types=[
          pltpu.VMEM((chunk,), jnp.int32),
          pltpu.VMEM((v_pad,), jnp.float32),
          pltpu.VMEM((chunk,), jnp.float32),
      ],
      compiler_params=pltpu.CompilerParams(needs_layout_passes=False),
  )
  def sc_gather(table_t_hbm, idx_hbm, out_hbm, idx_v, trow_v, out_v):
    wid = lax.axis_index("s") * NC + lax.axis_index("c")
    base = wid * chunk
    pltpu.sync_copy(idx_hbm.at[pl.ds(base, chunk)], idx_v)
    for h in range(num_heads):
      pltpu.sync_copy(table_t_hbm.at[h], trow_v)

      @plsc.parallel_loop(0, nvec, 1, unroll=8)
      def body(v):
        iv = idx_v[pl.ds(v * L, L)]
        out_v[pl.ds(v * L, L)] = plsc.load_gather(trow_v, [iv])
      pltpu.sync_copy(out_v, out_hbm.at[h, pl.ds(base, chunk)])

  return sc_gather


def kernel(relative_position_bias_table, relative_position_index):
  v, h = relative_position_bias_table.shape
  area = relative_position_index.shape[0]
  area2 = relative_position_index.size
  v_pad = -(-v // 8) * 8

  table_t = jnp.zeros((h, v_pad), jnp.float32)
  table_t = table_t.at[:, :v].set(relative_position_bias_table.T)
  idx_flat = relative_position_index.reshape(-1).astype(jnp.int32)

  out = _build_sc_gather(h, area2, v_pad)(table_t, idx_flat)
  return out.reshape(1, h, area, area)


# speedup vs baseline: 31.1951x; 1.2459x over previous
import functools

import jax
import jax.numpy as jnp
from jax import lax
from jax.experimental import pallas as pl
from jax.experimental.pallas import tpu as pltpu
from jax.experimental.pallas import tpu_sc as plsc

NC = 2
NS = 16
L = 16


@functools.lru_cache(maxsize=None)
def _build_sc_gather(num_heads: int, area2: int, v_pad: int):
  nw = NC * NS
  chunk = area2 // nw
  piece = 1024
  npiece = chunk // piece
  pvec = piece // L

  mesh = plsc.VectorSubcoreMesh(core_axis_name="c", subcore_axis_name="s")

  @functools.partial(
      pl.kernel,
      out_type=jax.ShapeDtypeStruct((num_heads, area2), jnp.float32),
      mesh=mesh,
      scratch_types=[
          pltpu.VMEM((chunk,), jnp.int32),
          pltpu.VMEM((num_heads, v_pad), jnp.float32),
          pltpu.VMEM((num_heads, piece), jnp.float32),
      ],
      compiler_params=pltpu.CompilerParams(needs_layout_passes=False),
  )
  def sc_gather(table_t_hbm, idx_hbm, out_hbm, idx_v, tab_v, out_v):
    wid = lax.axis_index("s") * NC + lax.axis_index("c")
    base = wid * chunk
    pltpu.sync_copy(idx_hbm.at[pl.ds(base, chunk)], idx_v)
    pltpu.sync_copy(table_t_hbm, tab_v)

    @pl.loop(0, npiece)
    def piece_loop(p):
      @plsc.parallel_loop(0, pvec, 1, unroll=4)
      def body(v):
        iv = idx_v[pl.ds(p * piece + v * L, L)]
        for h in range(num_heads):
          hv = jnp.full((L,), h, jnp.int32)
          out_v[h, pl.ds(v * L, L)] = plsc.load_gather(tab_v, [hv, iv])

      pltpu.sync_copy(out_v, out_hbm.at[:, pl.ds(base + p * piece, piece)])

  return sc_gather


def kernel(relative_position_bias_table, relative_position_index):
  v, h = relative_position_bias_table.shape
  area = relative_position_index.shape[0]
  area2 = relative_position_index.size
  v_pad = -(-v // 8) * 8

  table_t = jnp.zeros((h, v_pad), jnp.float32)
  table_t = table_t.at[:, :v].set(relative_position_bias_table.T)
  idx_flat = relative_position_index.reshape(-1).astype(jnp.int32)

  out = _build_sc_gather(h, area2, v_pad)(table_t, idx_flat)
  return out.reshape(1, h, area, area)


# speedup vs baseline: 36.6080x; 1.1735x over previous
import functools

import jax
import jax.numpy as jnp
from jax import lax
from jax.experimental import pallas as pl
from jax.experimental.pallas import tpu as pltpu
from jax.experimental.pallas import tpu_sc as plsc

NC = 2
NS = 16
L = 16


@functools.lru_cache(maxsize=None)
def _build_sc_gather(num_heads: int, area2: int, v_pad: int):
  nw = NC * NS
  chunk = area2 // nw
  piece = 512
  npiece = chunk // piece
  pvec = piece // L

  mesh = plsc.VectorSubcoreMesh(core_axis_name="c", subcore_axis_name="s")

  @functools.partial(
      pl.kernel,
      out_type=jax.ShapeDtypeStruct((num_heads, area2), jnp.float32),
      mesh=mesh,
      scratch_types=[
          pltpu.VMEM((chunk,), jnp.int32),
          pltpu.VMEM((num_heads, v_pad), jnp.float32),
          pltpu.VMEM((num_heads, piece), jnp.float32),
          pltpu.VMEM((num_heads, piece), jnp.float32),
          pltpu.SemaphoreType.DMA,
          pltpu.SemaphoreType.DMA,
      ],
      compiler_params=pltpu.CompilerParams(needs_layout_passes=False),
  )
  def sc_gather(table_t_hbm, idx_hbm, out_hbm, idx_v, tab_v, out_v0, out_v1,
                sem0, sem1):
    wid = lax.axis_index("s") * NC + lax.axis_index("c")
    base = wid * chunk
    pltpu.sync_copy(idx_hbm.at[pl.ds(base, chunk)], idx_v)
    pltpu.sync_copy(table_t_hbm, tab_v)
    bufs = (out_v0, out_v1)
    sems = (sem0, sem1)

    @pl.loop(0, npiece, step=2)
    def piece_loop(p):
      for k in range(2):
        buf, sem = bufs[k], sems[k]
        pp = p + k

        @pl.when(pp >= 2)
        def _wait_prev():
          pltpu.make_async_copy(
              buf, out_hbm.at[:, pl.ds(base + (pp - 2) * piece, piece)], sem
          ).wait()

        @plsc.parallel_loop(0, pvec, 1, unroll=4)
        def body(v):
          iv = idx_v[pl.ds(pp * piece + v * L, L)]
          for h in range(num_heads):
            hv = jnp.full((L,), h, jnp.int32)
            buf[h, pl.ds(v * L, L)] = plsc.load_gather(tab_v, [hv, iv])

        pltpu.async_copy(
            buf, out_hbm.at[:, pl.ds(base + pp * piece, piece)], sem)

    for k in range(2):
      pltpu.make_async_copy(
          bufs[k],
          out_hbm.at[:, pl.ds(base + (npiece - 2 + k) * piece, piece)],
          sems[k],
      ).wait()

  return sc_gather


def kernel(relative_position_bias_table, relative_position_index):
  v, h = relative_position_bias_table.shape
  area = relative_position_index.shape[0]
  area2 = relative_position_index.size
  v_pad = -(-v // 8) * 8

  table_t = jnp.zeros((h, v_pad), jnp.float32)
  table_t = table_t.at[:, :v].set(relative_position_bias_table.T)
  idx_flat = relative_position_index.reshape(-1).astype(jnp.int32)

  out = _build_sc_gather(h, area2, v_pad)(table_t, idx_flat)
  return out.reshape(1, h, area, area)


# speedup vs baseline: 67.3376x; 1.8394x over previous
import functools

import jax
import jax.numpy as jnp
from jax import lax
from jax.experimental import pallas as pl
from jax.experimental.pallas import tpu as pltpu
from jax.experimental.pallas import tpu_sc as plsc

NC = 2
NS = 16
L = 16


@functools.lru_cache(maxsize=None)
def _build_sc_gather(num_heads: int, area: int, v_pad: int):
  nw = NC * NS
  area2 = area * area
  chunk = area2 // nw
  piece = 512
  npiece = chunk // piece
  pvec = piece // L
  per_row = area // piece

  mesh = plsc.VectorSubcoreMesh(core_axis_name="c", subcore_axis_name="s")

  @functools.partial(
      pl.kernel,
      out_type=jax.ShapeDtypeStruct((1, num_heads, area, area), jnp.float32),
      mesh=mesh,
      scratch_types=[
          pltpu.VMEM((chunk,), jnp.int32),
          pltpu.VMEM((num_heads, v_pad), jnp.float32),
          pltpu.VMEM((num_heads, piece), jnp.float32),
          pltpu.VMEM((num_heads, piece), jnp.float32),
          pltpu.SemaphoreType.DMA,
          pltpu.SemaphoreType.DMA,
      ],
      compiler_params=pltpu.CompilerParams(needs_layout_passes=False),
  )
  def sc_gather(table_t_hbm, idx_hbm, out_hbm, idx_v, tab_v, out_v0, out_v1,
                sem0, sem1):
    wid = lax.axis_index("s") * NC + lax.axis_index("c")
    base = wid * chunk
    row0 = wid * (chunk // area)
    pltpu.sync_copy(idx_hbm.at[pl.ds(base, chunk)], idx_v)
    pltpu.sync_copy(table_t_hbm, tab_v)
    bufs = (out_v0, out_v1)
    sems = (sem0, sem1)

    def out_dst(pp):
      r = row0 + pp // per_row
      c = (pp % per_row) * piece
      return out_hbm.at[0, :, r, pl.ds(c, piece)]

    @pl.loop(0, npiece, step=2)
    def piece_loop(p):
      for k in range(2):
        buf, sem = bufs[k], sems[k]
        pp = p + k

        @pl.when(pp >= 2)
        def _wait_prev():
          pltpu.make_async_copy(buf, out_dst(pp - 2), sem).wait()

        @plsc.parallel_loop(0, pvec, 1, unroll=4)
        def body(v):
          iv = idx_v[pl.ds(pp * piece + v * L, L)]
          for h in range(num_heads):
            hv = jnp.full((L,), h, jnp.int32)
            buf[h, pl.ds(v * L, L)] = plsc.load_gather(tab_v, [hv, iv])

        pltpu.async_copy(buf, out_dst(pp), sem)

    for k in range(2):
      pltpu.make_async_copy(bufs[k], out_dst(npiece - 2 + k), sems[k]).wait()

  return sc_gather


def kernel(relative_position_bias_table, relative_position_index):
  v, h = relative_position_bias_table.shape
  area = relative_position_index.shape[0]
  area2 = relative_position_index.size
  v_pad = -(-v // 8) * 8

  table_t = jnp.zeros((h, v_pad), jnp.float32)
  table_t = table_t.at[:, :v].set(relative_position_bias_table.T)
  idx_flat = relative_position_index.reshape(-1).astype(jnp.int32)

  return _build_sc_gather(h, area, v_pad)(table_t, idx_flat)
